# baseline (speedup 1.0000x reference)
import functools

import jax
import jax.numpy as jnp
from jax import lax
from jax.experimental import pallas as pl
from jax.experimental.pallas import tpu as pltpu
from jax.experimental.pallas import tpu_sc as plsc

N = 50000
E = 800000
G = 64
F1P = 16
H = 300
HP = 384
NC = 2
NS = 16
NW = NC * NS
EPT = 25600
EPTOT = NW * EPT
OWN = 1632
NBIG = NW * OWN
SEG1 = 12800
SEG3 = 3200
BT = 128
XW = 256
OWN5 = 232
NP5 = 7
SEG5 = 3200
BT5 = 32
NBIG5 = NP5 * NW * OWN5

_mesh = plsc.VectorSubcoreMesh(core_axis_name="c", subcore_axis_name="s")


def _compact_seg(src_sl, dst_sl, scb, dcb, lo, nvec, keep_global, own=OWN):
    def scan_body(i, k):
        d16 = dst_sl[pl.ds(i * 16, 16)]
        dl = d16 - lo
        msk = (dl >= 0) & (dl < own)
        mi = msk.astype(jnp.int32)
        cs = plsc.cumsum(mi)
        pos = jnp.where(msk, k - 1 + cs, dcb.shape[0] - 8)
        if src_sl is not None:
            s16 = src_sl[pl.ds(i * 16, 16)]
            plsc.store_scatter(scb, [pos], s16)
        plsc.store_scatter(dcb, [pos], d16 if keep_global else dl)
        return k + jnp.sum(mi)

    return lax.fori_loop(0, nvec, scan_body, 0)


@functools.partial(
    pl.kernel,
    out_type=jax.ShapeDtypeStruct((NBIG * F1P,), jnp.float32),
    mesh=_mesh,
    scratch_types=[
        pltpu.VMEM((SEG1,), jnp.int32),
        pltpu.VMEM((SEG1 + 144,), jnp.int32),
        pltpu.VMEM(((OWN + 9) * F1P,), jnp.float32),
    ],
    compiler_params=pltpu.CompilerParams(needs_layout_passes=False),
)
def _deg_kernel(dst_hbm, out_hbm, dst_v, dcb, acc):
    cid = lax.axis_index("c")
    sid = lax.axis_index("s")
    w = cid * NS + sid
    lo = w * OWN

    def zero_body(r, c):
        acc[pl.ds(r * 16, 16)] = jnp.zeros((16,), jnp.float32)
        return c

    lax.fori_loop(0, (OWN + 1) * F1P // 16, zero_body, 0)

    ones = jnp.ones((16,), jnp.float32)
    iota = lax.broadcasted_iota(jnp.int32, (16,), 0)

    def seg_loop(seg, carry):
        pltpu.sync_copy(dst_hbm.at[pl.ds(seg * SEG1, SEG1)], dst_v)
        k = _compact_seg(None, dst_v, None, dcb, lo, SEG1 // 16, False)

        def edge_body(e, c):
            dlb = plsc.load_gather(dcb, [jnp.full((16,), e, jnp.int32)])
            plsc.addupdate_scatter(acc, [dlb * F1P + iota], ones)
            return c

        lax.fori_loop(0, k, edge_body, 0)
        return carry

    lax.fori_loop(0, EPTOT // SEG1, seg_loop, 0)
    pltpu.sync_copy(acc.at[pl.ds(0, OWN * F1P)],
                    out_hbm.at[pl.ds(lo * F1P, OWN * F1P)])


@functools.partial(
    pl.kernel,
    out_type=jax.ShapeDtypeStruct((NBIG * F1P,), jnp.float32),
    mesh=_mesh,
    scratch_types=[
        pltpu.VMEM((SEG3,), jnp.int32),
        pltpu.VMEM((SEG3,), jnp.int32),
        pltpu.VMEM((SEG3 + 144,), jnp.int32),
        pltpu.VMEM((SEG3 + 144,), jnp.int32),
        pltpu.VMEM((BT, XW), jnp.float32),
        pltpu.VMEM(((OWN + 9) * F1P,), jnp.float32),
        pltpu.SemaphoreType.DMA,
    ],
    compiler_params=pltpu.CompilerParams(needs_layout_passes=False),
)
def _agg1_kernel(src_hbm, dst_hbm, xs_hbm, out_hbm,
                 src_v, dst_v, scb, dcb, rows_v, acc, gsem):
    cid = lax.axis_index("c")
    sid = lax.axis_index("s")
    w = cid * NS + sid
    lo = w * OWN

    def zero_body(r, c):
        acc[pl.ds(r * 16, 16)] = jnp.zeros((16,), jnp.float32)
        return c

    lax.fori_loop(0, (OWN + 1) * F1P // 16, zero_body, 0)

    iota = lax.broadcasted_iota(jnp.int32, (16,), 0)

    def seg_loop(seg, carry):
        pltpu.sync_copy(src_hbm.at[pl.ds(seg * SEG3, SEG3)], src_v)
        pltpu.sync_copy(dst_hbm.at[pl.ds(seg * SEG3, SEG3)], dst_v)
        k = _compact_seg(src_v, dst_v, scb, dcb, lo, SEG3 // 16, False)

        for m in range(BT // 16):
            scb[pl.ds(k + m * 16, 16)] = jnp.zeros((16,), jnp.int32)

        def batch_body(j, c):
            base = j * BT

            @pl.when(base < k)
            def _():
                pltpu.async_copy(xs_hbm.at[scb.at[pl.ds(j * BT, BT)]],
                                 rows_v, gsem).wait()
                nloc = jnp.minimum(k - base, BT)

                def edge_body(e, c2):
                    eb = jnp.full((16,), base + e, jnp.int32)
                    dlb = plsc.load_gather(dcb, [eb])
                    val = plsc.load_gather(
                        rows_v, [jnp.full((16,), e, jnp.int32), iota])
                    plsc.addupdate_scatter(acc, [dlb * F1P + iota], val)
                    return c2

                lax.fori_loop(0, nloc, edge_body, 0)

            return c

        lax.fori_loop(0, SEG3 // BT, batch_body, 0)
        return carry

    lax.fori_loop(0, EPTOT // SEG3, seg_loop, 0)
    pltpu.sync_copy(acc.at[pl.ds(0, OWN * F1P)],
                    out_hbm.at[pl.ds(lo * F1P, OWN * F1P)])


@functools.partial(
    pl.kernel,
    out_type=jax.ShapeDtypeStruct((NBIG5 * HP,), jnp.float32),
    mesh=_mesh,
    scratch_types=[
        pltpu.VMEM((SEG5,), jnp.int32),
        pltpu.VMEM((SEG5,), jnp.int32),
        pltpu.VMEM((SEG5 + 144,), jnp.int32),
        pltpu.VMEM((SEG5 + 144,), jnp.int32),
        pltpu.VMEM((BT5, HP), jnp.float32),
        pltpu.VMEM(((OWN5 + 1) * HP,), jnp.float32),
        pltpu.SemaphoreType.DMA,
    ],
    compiler_params=pltpu.CompilerParams(needs_layout_passes=False),
)
def _agg2_kernel(src_hbm, dst_hbm, h1s_hbm, out_hbm,
                 src_v, dst_v, scb, dcb, rows_v, acc, gsem):
    cid = lax.axis_index("c")
    sid = lax.axis_index("s")
    w = cid * NS + sid

    iota = lax.broadcasted_iota(jnp.int32, (16,), 0)

    def one_pass(p, pc):
        lo = p * (NW * OWN5) + w * OWN5

        def zero_body(r, c):
            acc[pl.ds(r * 16, 16)] = jnp.zeros((16,), jnp.float32)
            return c

        lax.fori_loop(0, (OWN5 + 1) * HP // 16, zero_body, 0)

        def seg_loop(seg, carry):
            pltpu.sync_copy(src_hbm.at[pl.ds(seg * SEG5, SEG5)], src_v)
            pltpu.sync_copy(dst_hbm.at[pl.ds(seg * SEG5, SEG5)], dst_v)
            k = _compact_seg(src_v, dst_v, scb, dcb, lo, SEG5 // 16, False,
                             own=OWN5)

            for m in range(BT5 // 16):
                scb[pl.ds(k + m * 16, 16)] = jnp.zeros((16,), jnp.int32)

            def batch_body(j, c):
                base = j * BT5

                @pl.when(base < k)
                def _():
                    pltpu.async_copy(h1s_hbm.at[scb.at[pl.ds(j * BT5, BT5)]],
                                     rows_v, gsem).wait()
                    nloc = jnp.minimum(k - base, BT5)

                    def edge_body(e, c2):
                        dlb = plsc.load_gather(
                            dcb, [jnp.full((16,), base + e, jnp.int32)])
                        pbase = dlb * HP
                        for cc in range(HP // 16):
                            val = plsc.load_gather(
                                rows_v,
                                [jnp.full((16,), e, jnp.int32),
                                 cc * 16 + iota])
                            plsc.addupdate_scatter(
                                acc, [pbase + cc * 16 + iota], val)
                        return c2

                    lax.fori_loop(0, nloc, edge_body, 0)

                return c

            lax.fori_loop(0, SEG5 // BT5, batch_body, 0)
            return carry

        lax.fori_loop(0, EPTOT // SEG5, seg_loop, 0)
        pltpu.sync_copy(acc.at[pl.ds(0, OWN5 * HP)],
                        out_hbm.at[pl.ds(lo * HP, OWN5 * HP)])
        return pc

    lax.fori_loop(0, NP5, one_pass, 0)


def _k2_body(p_ref, x_ref, xs_ref, dis_ref):
    deg = p_ref[:, 0:1] + 1.0
    dis = lax.rsqrt(deg)
    dis_ref[...] = dis
    xs_ref[...] = x_ref[...] * dis


def _k4_body(p_ref, x_ref, dis_ref, w1_ref, b1_ref, h1s_ref):
    dis = dis_ref[...]
    z = (p_ref[...] + x_ref[...] * dis) * dis
    h1 = jnp.dot(z, w1_ref[...], preferred_element_type=jnp.float32)
    h1 = jnp.maximum(h1 + b1_ref[...], 0.0)
    h1s_ref[...] = h1 * dis


def _k6_body(p_ref, h1s_ref, dis_ref, bat_ref, w2_ref, b2_ref,
             wf_ref, bf_ref, out_ref, sums_ref, cnts_ref):
    i = pl.program_id(0)
    nb = pl.num_programs(0)
    z = (p_ref[...] + h1s_ref[...]) * dis_ref[...]
    h2 = jnp.dot(z, w2_ref[...], preferred_element_type=jnp.float32)
    h2 = jnp.maximum(h2 + b2_ref[...], 0.0)
    s = (bat_ref[...] == lax.broadcasted_iota(jnp.int32, (bat_ref.shape[0], G), 1))
    s = s.astype(jnp.float32)
    ps = lax.dot_general(s, h2, (((0,), (0,)), ((), ())),
                         preferred_element_type=jnp.float32)
    pc = lax.dot_general(s, jnp.ones((s.shape[0], 1), jnp.float32),
                         (((0,), (0,)), ((), ())),
                         preferred_element_type=jnp.float32)

    @pl.when(i == 0)
    def _():
        sums_ref[...] = jnp.zeros_like(sums_ref)
        cnts_ref[...] = jnp.zeros_like(cnts_ref)

    sums_ref[...] += ps
    cnts_ref[...] += pc

    @pl.when(i == nb - 1)
    def _():
        pooled = sums_ref[...] / jnp.maximum(cnts_ref[...], 1.0)
        out_ref[...] = jnp.dot(pooled, wf_ref[...],
                               preferred_element_type=jnp.float32) + bf_ref[...]


_BLK = 2000


def kernel(x, edge_index, batch, W1, b1, W2, b2, Wf, bf):
    ei = edge_index.astype(jnp.int32)
    src = jnp.pad(ei[0], (0, EPTOT - E))
    dst = jnp.pad(ei[1], (0, EPTOT - E), constant_values=N)
    xpad = jnp.pad(x, ((0, 0), (0, F1P - x.shape[1])))
    xpadw = jnp.pad(x, ((0, 0), (0, XW - x.shape[1])))
    W1p = jnp.pad(W1, ((0, F1P - W1.shape[0]), (0, HP - H)))
    b1p = jnp.pad(b1, (0, HP - H)).reshape(1, HP)
    W2p = jnp.pad(W2, ((0, HP - H), (0, HP - H)))
    b2p = jnp.pad(b2, (0, HP - H)).reshape(1, HP)
    Wfp = jnp.pad(Wf, ((0, HP - H), (0, 128 - Wf.shape[1])))
    bfp = jnp.pad(bf, (0, 128 - bf.shape[0])).reshape(1, 128)
    bat = batch.astype(jnp.int32).reshape(N, 1)

    nblk = N // _BLK

    degp = _deg_kernel(dst).reshape(NBIG, F1P)

    xs, dis = pl.pallas_call(
        _k2_body,
        grid=(nblk,),
        in_specs=[
            pl.BlockSpec((_BLK, F1P), lambda i: (i, 0)),
            pl.BlockSpec((_BLK, XW), lambda i: (i, 0)),
        ],
        out_specs=[
            pl.BlockSpec((_BLK, XW), lambda i: (i, 0)),
            pl.BlockSpec((_BLK, 1), lambda i: (i, 0)),
        ],
        out_shape=[
            jax.ShapeDtypeStruct((N, XW), jnp.float32),
            jax.ShapeDtypeStruct((N, 1), jnp.float32),
        ],
    )(degp, xpadw)

    agg1 = _agg1_kernel(src, dst, xs).reshape(NBIG, F1P)

    h1s = pl.pallas_call(
        _k4_body,
        grid=(nblk,),
        in_specs=[
            pl.BlockSpec((_BLK, F1P), lambda i: (i, 0)),
            pl.BlockSpec((_BLK, F1P), lambda i: (i, 0)),
            pl.BlockSpec((_BLK, 1), lambda i: (i, 0)),
            pl.BlockSpec((F1P, HP), lambda i: (0, 0)),
            pl.BlockSpec((1, HP), lambda i: (0, 0)),
        ],
        out_specs=pl.BlockSpec((_BLK, HP), lambda i: (i, 0)),
        out_shape=jax.ShapeDtypeStruct((N, HP), jnp.float32),
    )(agg1, xpad, dis, W1p, b1p)

    agg2 = _agg2_kernel(src, dst, h1s).reshape(NBIG5, HP)

    out = pl.pallas_call(
        _k6_body,
        grid=(nblk,),
        in_specs=[
            pl.BlockSpec((_BLK, HP), lambda i: (i, 0)),
            pl.BlockSpec((_BLK, HP), lambda i: (i, 0)),
            pl.BlockSpec((_BLK, 1), lambda i: (i, 0)),
            pl.BlockSpec((_BLK, 1), lambda i: (i, 0)),
            pl.BlockSpec((HP, HP), lambda i: (0, 0)),
            pl.BlockSpec((1, HP), lambda i: (0, 0)),
            pl.BlockSpec((HP, 128), lambda i: (0, 0)),
            pl.BlockSpec((1, 128), lambda i: (0, 0)),
        ],
        out_specs=pl.BlockSpec((G, 128), lambda i: (0, 0)),
        out_shape=jax.ShapeDtypeStruct((G, 128), jnp.float32),
        scratch_shapes=[
            pltpu.VMEM((G, HP), jnp.float32),
            pltpu.VMEM((G, 1), jnp.float32),
        ],
    )(agg2, h1s, dis, bat, W2p, b2p, Wfp, bfp)

    return out[:, : Wf.shape[1]]
